# baseline (speedup 1.0000x reference)
import functools

import jax
import jax.numpy as jnp
import numpy as np
from jax.experimental import pallas as pl
from jax.experimental.pallas import tpu as pltpu

_N_TOK = 448
_SEQ = 2048
_D = 768
_LANES = 128
_ROWS = _N_TOK * (_SEQ // _LANES)
_EPS = float(np.finfo(np.float64).eps)

_TB = 8
_BLK = _TB * (_SEQ // _LANES)


def _body(x0_ref, x1_ref, x2_ref, lg_ref, w0_ref, w1_ref, w2_ref, bv_ref,
          out_ref, scr_ref):
    lanes = ((0, 16), (16, 48), (48, 112))
    for e, (x_ref, w_ref, ngrp) in enumerate(((x0_ref, w0_ref, 8),
                                              (x1_ref, w1_ref, 4),
                                              (x2_ref, w2_ref, 2))):
        L = ngrp * (_SEQ // _LANES)
        x = x_ref[...].reshape(_TB * L, _D).astype(jnp.bfloat16)
        y = jnp.dot(x, w_ref[...], preferred_element_type=jnp.float32)
        lo, hi = lanes[e]
        scr_ref[0:_TB * L, lo:hi] = y
    lg = lg_ref[...]
    acc = None
    for e, ngrp in enumerate((8, 4, 2)):
        lo, hi = lanes[e]
        parts = [
            scr_ref[pl.Slice(k, _BLK, ngrp), :][:, lo:hi]
            for k in range(ngrp)
        ]
        y = jnp.concatenate(parts, axis=1)
        z = jnp.exp(y + bv_ref[e:e + 1, :] + lg[:, e:e + 1])
        acc = z if acc is None else acc + z
    out_ref[...] = jnp.log(jnp.where(acc == 0, jnp.float32(_EPS), acc))


@functools.partial(jax.jit, static_argnames=())
def kernel(xs0, xs1, xs2, gates, x_dec, W0, b0, W1, b1, W2, b2):
    del x_dec
    W0t = W0.T.astype(jnp.bfloat16)
    W1t = W1.T.astype(jnp.bfloat16)
    W2t = W2.T.astype(jnp.bfloat16)
    LG = jnp.repeat(jnp.log(gates), _SEQ // _LANES, axis=0)
    BV = jnp.zeros((8, _LANES), jnp.float32)
    BV = BV.at[0].set(jnp.tile(b0, _LANES // 16))
    BV = BV.at[1].set(jnp.tile(b1, _LANES // 32))
    BV = BV.at[2].set(jnp.tile(b2, _LANES // 64))

    grid = (_N_TOK // _TB,)
    out = pl.pallas_call(
        _body,
        grid=grid,
        in_specs=[
            pl.BlockSpec((_TB, 128, _D), lambda i: (i, 0, 0)),
            pl.BlockSpec((_TB, 64, _D), lambda i: (i, 0, 0)),
            pl.BlockSpec((_TB, 32, _D), lambda i: (i, 0, 0)),
            pl.BlockSpec((_BLK, 3), lambda i: (i, 0)),
            pl.BlockSpec((_D, 16), lambda i: (0, 0)),
            pl.BlockSpec((_D, 32), lambda i: (0, 0)),
            pl.BlockSpec((_D, 64), lambda i: (0, 0)),
            pl.BlockSpec((8, _LANES), lambda i: (0, 0)),
        ],
        out_specs=pl.BlockSpec((_BLK, _LANES), lambda i: (i, 0)),
        out_shape=jax.ShapeDtypeStruct((_ROWS, _LANES), jnp.float32),
        scratch_shapes=[pltpu.VMEM((_TB * 128, _LANES), jnp.float32)],
        compiler_params=pltpu.CompilerParams(
            dimension_semantics=("parallel",)),
    )(xs0, xs1, xs2, LG, W0t, W1t, W2t, BV)

    B = _N_TOK // 14
    return out.reshape(B, 14, _SEQ).transpose(0, 2, 1)


# speedup vs baseline: 1.1462x; 1.1462x over previous
import functools

import jax
import jax.numpy as jnp
import numpy as np
from jax.experimental import pallas as pl
from jax.experimental.pallas import tpu as pltpu

_N_TOK = 448
_SEQ = 2048
_D = 768
_LANES = 128
_ROWS = _N_TOK * (_SEQ // _LANES)
_EPS = float(np.finfo(np.float64).eps)

_TB = 16
_BLK = _TB * (_SEQ // _LANES)


def _body(x0_ref, x1_ref, x2_ref, lg_ref, w0_ref, w1_ref, w2_ref, bv_ref,
          out_ref, scr_ref):
    lanes = ((0, 16), (16, 48), (48, 112))
    for e, (x_ref, w_ref, ngrp) in enumerate(((x0_ref, w0_ref, 8),
                                              (x1_ref, w1_ref, 4),
                                              (x2_ref, w2_ref, 2))):
        L = ngrp * (_SEQ // _LANES)
        x = x_ref[...].reshape(_TB * L, _D).astype(jnp.bfloat16)
        y = jnp.dot(x, w_ref[...], preferred_element_type=jnp.float32)
        lo, hi = lanes[e]
        scr_ref[0:_TB * L, lo:hi] = y
    lg = lg_ref[...]
    acc = None
    for e, ngrp in enumerate((8, 4, 2)):
        lo, hi = lanes[e]
        parts = [
            scr_ref[pl.Slice(k, _BLK, ngrp), :][:, lo:hi]
            for k in range(ngrp)
        ]
        y = jnp.concatenate(parts, axis=1)
        z = jnp.exp(y + bv_ref[e:e + 1, :] + lg[:, e:e + 1])
        acc = z if acc is None else acc + z
    out_ref[...] = jnp.log(jnp.where(acc == 0, jnp.float32(_EPS), acc))


@functools.partial(jax.jit, static_argnames=())
def kernel(xs0, xs1, xs2, gates, x_dec, W0, b0, W1, b1, W2, b2):
    del x_dec
    W0t = W0.T.astype(jnp.bfloat16)
    W1t = W1.T.astype(jnp.bfloat16)
    W2t = W2.T.astype(jnp.bfloat16)
    LG = jnp.repeat(jnp.log(gates), _SEQ // _LANES, axis=0)
    BV = jnp.zeros((8, _LANES), jnp.float32)
    BV = BV.at[0].set(jnp.tile(b0, _LANES // 16))
    BV = BV.at[1].set(jnp.tile(b1, _LANES // 32))
    BV = BV.at[2].set(jnp.tile(b2, _LANES // 64))

    grid = (_N_TOK // _TB,)
    out = pl.pallas_call(
        _body,
        grid=grid,
        in_specs=[
            pl.BlockSpec((_TB, 128, _D), lambda i: (i, 0, 0)),
            pl.BlockSpec((_TB, 64, _D), lambda i: (i, 0, 0)),
            pl.BlockSpec((_TB, 32, _D), lambda i: (i, 0, 0)),
            pl.BlockSpec((_BLK, 3), lambda i: (i, 0)),
            pl.BlockSpec((_D, 16), lambda i: (0, 0)),
            pl.BlockSpec((_D, 32), lambda i: (0, 0)),
            pl.BlockSpec((_D, 64), lambda i: (0, 0)),
            pl.BlockSpec((8, _LANES), lambda i: (0, 0)),
        ],
        out_specs=pl.BlockSpec((_BLK, _LANES), lambda i: (i, 0)),
        out_shape=jax.ShapeDtypeStruct((_ROWS, _LANES), jnp.float32),
        scratch_shapes=[pltpu.VMEM((_TB * 128, _LANES), jnp.float32)],
        compiler_params=pltpu.CompilerParams(
            dimension_semantics=("parallel",)),
    )(xs0, xs1, xs2, LG, W0t, W1t, W2t, BV)

    B = _N_TOK // 14
    return out.reshape(B, 14, _SEQ).transpose(0, 2, 1)


# speedup vs baseline: 1.2101x; 1.0557x over previous
import functools

import jax
import jax.numpy as jnp
import numpy as np
from jax.experimental import pallas as pl
from jax.experimental.pallas import tpu as pltpu

_N_TOK = 448
_SEQ = 2048
_D = 768
_LANES = 128
_ROWS = _N_TOK * (_SEQ // _LANES)
_EPS = float(np.finfo(np.float64).eps)

_TB = 16
_BLK = _TB * (_SEQ // _LANES)


_DIMNUM = (((1,), (1,)), ((), ()))


def _body(x0_ref, x1_ref, x2_ref, g_ref, w0_ref, w1_ref, w2_ref,
          bv0_ref, bv1_ref, bv2_ref, out_ref, scr_ref):
    lanes = ((0, 16), (16, 48), (48, 112))
    for e, (x_ref, w_ref, ngrp) in enumerate(((x0_ref, w0_ref, 8),
                                              (x1_ref, w1_ref, 4),
                                              (x2_ref, w2_ref, 2))):
        L = ngrp * (_SEQ // _LANES)
        x = x_ref[...].reshape(_TB * L, _D).astype(jnp.bfloat16)
        y = jax.lax.dot_general(x, w_ref[...].astype(jnp.bfloat16), _DIMNUM,
                                preferred_element_type=jnp.float32)
        lo, hi = lanes[e]
        scr_ref[0:_TB * L, lo:hi] = y
    lg = jnp.repeat(jnp.log(g_ref[...]), _SEQ // _LANES, axis=0)
    acc = None
    for e, (bv_ref, ngrp) in enumerate(((bv0_ref, 8), (bv1_ref, 4),
                                        (bv2_ref, 2))):
        lo, hi = lanes[e]
        parts = [
            scr_ref[pl.Slice(k, _BLK, ngrp), :][:, lo:hi]
            for k in range(ngrp)
        ]
        y = jnp.concatenate(parts, axis=1)
        z = jnp.exp(y + bv_ref[...] + lg[:, e:e + 1])
        acc = z if acc is None else acc + z
    out_ref[...] = jnp.log(jnp.where(acc == 0, jnp.float32(_EPS), acc))


@functools.partial(jax.jit, static_argnames=())
def kernel(xs0, xs1, xs2, gates, x_dec, W0, b0, W1, b1, W2, b2):
    del x_dec
    bv0 = jnp.tile(b0, _LANES // 16)[None, :]
    bv1 = jnp.tile(b1, _LANES // 32)[None, :]
    bv2 = jnp.tile(b2, _LANES // 64)[None, :]

    grid = (_N_TOK // _TB,)
    out = pl.pallas_call(
        _body,
        grid=grid,
        in_specs=[
            pl.BlockSpec((_TB, 128, _D), lambda i: (i, 0, 0)),
            pl.BlockSpec((_TB, 64, _D), lambda i: (i, 0, 0)),
            pl.BlockSpec((_TB, 32, _D), lambda i: (i, 0, 0)),
            pl.BlockSpec((_TB, 3), lambda i: (i, 0)),
            pl.BlockSpec((16, _D), lambda i: (0, 0)),
            pl.BlockSpec((32, _D), lambda i: (0, 0)),
            pl.BlockSpec((64, _D), lambda i: (0, 0)),
            pl.BlockSpec((1, _LANES), lambda i: (0, 0)),
            pl.BlockSpec((1, _LANES), lambda i: (0, 0)),
            pl.BlockSpec((1, _LANES), lambda i: (0, 0)),
        ],
        out_specs=pl.BlockSpec((_BLK, _LANES), lambda i: (i, 0)),
        out_shape=jax.ShapeDtypeStruct((_ROWS, _LANES), jnp.float32),
        scratch_shapes=[pltpu.VMEM((_TB * 128, _LANES), jnp.float32)],
        compiler_params=pltpu.CompilerParams(
            dimension_semantics=("parallel",)),
    )(xs0, xs1, xs2, gates, W0, W1, W2, bv0, bv1, bv2)

    B = _N_TOK // 14
    return out.reshape(B, 14, _SEQ).transpose(0, 2, 1)
